# Initial kernel scaffold; baseline (speedup 1.0000x reference)
import jax
import jax.numpy as jnp
from jax.experimental import pallas as pl


def kernel(x, edge_attr, edge_index, edge_weight, laplacian_index, laplacian_weight, W_emb_node, W_convs_node, W_out_node, W_emb_edge, W_convs_edge, W_out_edge):
    raise NotImplementedError("write your pallas kernel here")


# speedup vs baseline: 1.2317x; 1.2317x over previous
import functools

import jax
import jax.numpy as jnp
from jax import lax
from jax.experimental import pallas as pl
from jax.experimental.pallas import tpu as pltpu
from jax.experimental.pallas import tpu_sc as plsc

_NC = 2
_NS = 16
_NW = _NC * _NS

_N_NODES = 10000
_N_PAD = 10240
_N_EDGES = 320000
_HID = 128
_W = 256
_E_PAD = 327680


def _mm_body(x_ref, w_ref, o_ref, *, act):
    y = jnp.dot(x_ref[...], w_ref[...], preferred_element_type=jnp.float32)
    if act:
        y = jnp.where(y >= 0, y, 0.2 * y)
    o_ref[...] = y


def _mm(x, w, act=False, block=2000):
    n, din = x.shape
    dout = w.shape[1]
    grid = n // block
    return pl.pallas_call(
        functools.partial(_mm_body, act=act),
        grid=(grid,),
        in_specs=[
            pl.BlockSpec((block, din), lambda i: (i, 0)),
            pl.BlockSpec((din, dout), lambda i: (0, 0)),
        ],
        out_specs=pl.BlockSpec((block, dout), lambda i: (i, 0)),
        out_shape=jax.ShapeDtypeStruct((n, dout), jnp.float32),
    )(x, w)


def _mm2_body(a_ref, b_ref, w_ref, o_ref):
    y = jnp.dot(a_ref[...] + b_ref[...], w_ref[...],
                preferred_element_type=jnp.float32)
    o_ref[...] = y


def _mm2(a, b, w, block=2048):
    n, din = a.shape
    dout = w.shape[1]
    grid = n // block
    return pl.pallas_call(
        _mm2_body,
        grid=(grid,),
        in_specs=[
            pl.BlockSpec((block, din), lambda i: (i, 0)),
            pl.BlockSpec((block, din), lambda i: (i, 0)),
            pl.BlockSpec((din, dout), lambda i: (0, 0)),
        ],
        out_specs=pl.BlockSpec((block, dout), lambda i: (i, 0)),
        out_shape=jax.ShapeDtypeStruct((n, dout), jnp.float32),
    )(a, b, w)


def _node_segsum_body(h_hbm, src_hbm, dst_hbm, w_hbm, out_hbm,
                      src_v, dst_v, w_v, rows_v, zrow_v, acc_sh, sem):
    c = lax.axis_index("c")
    s = lax.axis_index("s")
    wid = s * _NC + c
    n_win = _E_PAD // _NW // _W

    def _zb(r, _):
        for j in range(8):
            zrow_v[r, pl.ds(j * 16, 16)] = jnp.zeros((16,), jnp.float32)
        return 0
    lax.fori_loop(0, 80, _zb, 0)
    stripe = s * (_N_PAD // _NS)
    for t in range(8):
        pltpu.sync_copy(zrow_v, acc_sh.at[pl.ds(stripe + t * 80, 80)])
    plsc.subcore_barrier()

    def _win(wi, _):
        base = wid * (_E_PAD // _NW) + wi * _W
        pltpu.sync_copy(src_hbm.at[pl.ds(base, _W)], src_v)
        pltpu.sync_copy(dst_hbm.at[pl.ds(base, _W)], dst_v)
        pltpu.sync_copy(w_hbm.at[pl.ds(base, _W)], w_v)
        pltpu.async_copy(h_hbm.at[src_v], rows_v, sem).wait()

        def _row16(r16, _):
            wvec = w_v[pl.ds(r16 * 16, 16)]
            for l in range(16):
                wb = jnp.full((16,), wvec[l], jnp.float32)
                r = r16 * 16 + l
                for j in range(8):
                    sl = pl.ds(j * 16, 16)
                    rows_v[r, sl] = rows_v[r, sl] * wb
            return 0
        lax.fori_loop(0, _W // 16, _row16, 0)
        pltpu.sync_copy(rows_v, acc_sh.at[dst_v], add=True)
        return 0
    lax.fori_loop(0, n_win, _win, 0)

    plsc.subcore_barrier()
    rows_per_tile = _N_PAD // _NS
    pltpu.sync_copy(acc_sh.at[pl.ds(stripe, rows_per_tile)],
                    out_hbm.at[c, pl.ds(stripe, rows_per_tile)])


def _node_segsum(h, src, dst, w):
    mesh = plsc.VectorSubcoreMesh(core_axis_name="c", subcore_axis_name="s")
    return pl.kernel(
        _node_segsum_body,
        out_type=jax.ShapeDtypeStruct((_NC, _N_PAD, _HID), jnp.float32),
        mesh=mesh,
        scratch_types=[
            pltpu.VMEM((_W,), jnp.int32),
            pltpu.VMEM((_W,), jnp.int32),
            pltpu.VMEM((_W,), jnp.float32),
            pltpu.VMEM((_W, _HID), jnp.float32),
            pltpu.VMEM((80, _HID), jnp.float32),
            pltpu.VMEM_SHARED((_N_PAD, _HID), jnp.float32),
            pltpu.SemaphoreType.DMA,
        ],
    )(h, src, dst, w)


def _node_cochain(x, index, weight, W_emb, W_convs, W_out):
    h = _mm(x, W_emb, act=True)
    h = jnp.concatenate(
        [h, jnp.zeros((_N_PAD - _N_NODES, _HID), jnp.float32)], axis=0)
    npad = _E_PAD - _N_EDGES
    fill = jnp.arange(npad, dtype=jnp.int32)
    src = jnp.concatenate([index[0], fill % _N_NODES])
    dst = jnp.concatenate([index[1], fill % _N_PAD])
    w = jnp.concatenate([weight, jnp.zeros((npad,), jnp.float32)])
    for l in range(W_convs.shape[0] - 1):
        p = _node_segsum(h, src, dst, w)
        h = _mm2(p[0], p[1], W_convs[l])
    return _mm(h, W_out, block=2048)[:_N_NODES]


def _edge_cochain(feat, index, weight, W_emb, W_convs, W_out):
    h = _mm(feat, W_emb, act=True, block=8000)
    src, dst = index[0], index[1]
    for l in range(W_convs.shape[0] - 1):
        msg = h[src] * weight[:, None]
        agg = jax.ops.segment_sum(msg, dst, num_segments=_N_EDGES)
        h = _mm(agg, W_convs[l], block=8000)
    return _mm(h, W_out, block=8000)


def kernel(x, edge_attr, edge_index, edge_weight, laplacian_index,
           laplacian_weight, W_emb_node, W_convs_node, W_out_node,
           W_emb_edge, W_convs_edge, W_out_edge):
    node_out = _node_cochain(x, edge_index, edge_weight, W_emb_node,
                             W_convs_node, W_out_node)
    edge_out = _edge_cochain(edge_attr, laplacian_index, laplacian_weight,
                             W_emb_edge, W_convs_edge, W_out_edge)
    return (node_out, edge_out)


# speedup vs baseline: 3.0370x; 2.4657x over previous
import functools

import jax
import jax.numpy as jnp
from jax import lax
from jax.experimental import pallas as pl
from jax.experimental.pallas import tpu as pltpu
from jax.experimental.pallas import tpu_sc as plsc

_NC = 2
_NS = 16
_NW = _NC * _NS

_N_NODES = 10000
_N_PAD = 10240
_N_EDGES = 320000
_HID = 128
_W = 128
_NBW = 8
_E_PAD = 327680


def _mm_body(x_ref, w_ref, o_ref, *, act):
    y = jnp.dot(x_ref[...], w_ref[...], preferred_element_type=jnp.float32)
    if act:
        y = jnp.where(y >= 0, y, 0.2 * y)
    o_ref[...] = y


def _mm(x, w, act=False, block=2000):
    n, din = x.shape
    dout = w.shape[1]
    grid = n // block
    return pl.pallas_call(
        functools.partial(_mm_body, act=act),
        grid=(grid,),
        in_specs=[
            pl.BlockSpec((block, din), lambda i: (i, 0)),
            pl.BlockSpec((din, dout), lambda i: (0, 0)),
        ],
        out_specs=pl.BlockSpec((block, dout), lambda i: (i, 0)),
        out_shape=jax.ShapeDtypeStruct((n, dout), jnp.float32),
    )(x, w)


def _mm2_body(a_ref, b_ref, w_ref, o_ref):
    y = jnp.dot(a_ref[...] + b_ref[...], w_ref[...],
                preferred_element_type=jnp.float32)
    o_ref[...] = y


def _mm2(a, b, w, block=2048):
    n, din = a.shape
    dout = w.shape[1]
    grid = n // block
    return pl.pallas_call(
        _mm2_body,
        grid=(grid,),
        in_specs=[
            pl.BlockSpec((block, din), lambda i: (i, 0)),
            pl.BlockSpec((block, din), lambda i: (i, 0)),
            pl.BlockSpec((din, dout), lambda i: (0, 0)),
        ],
        out_specs=pl.BlockSpec((block, dout), lambda i: (i, 0)),
        out_shape=jax.ShapeDtypeStruct((n, dout), jnp.float32),
    )(a, b, w)


def _zero_rows(rows_v):
    def _zb(r, _):
        for j in range(_HID // 16):
            rows_v[r, pl.ds(j * 16, 16)] = jnp.zeros((16,), jnp.float32)
        return 0
    lax.fori_loop(0, _W, _zb, 0)


def _gather_dma(h_hbm, sb_v, rows_v, sem, slot):
    idxr = sb_v.at[pl.ds(slot * _W, _W)]
    return pltpu.make_async_copy(h_hbm.at[idxr], rows_v, sem)


def _scale_rows(rows_v, wb_v, slot):
    def _r16(r16, _):
        wvec = wb_v[pl.ds(slot * _W + r16 * 16, 16)]
        for l in range(16):
            wf = jnp.full((16,), wvec[l], jnp.float32)
            r = r16 * 16 + l
            for j in range(_HID // 16):
                sl = pl.ds(j * 16, 16)
                rows_v[r, sl] = rows_v[r, sl] * wf
        return 0
    lax.fori_loop(0, _W // 16, _r16, 0)


def _node_segsum_body(h_hbm, src_hbm, dst_hbm, w_hbm, out_hbm,
                      sb_v, db_v, wb_v, dst1_v, rows0_v, rows1_v, acc_sh,
                      sem0, sem1):
    c = lax.axis_index("c")
    s = lax.axis_index("s")
    wid = s * _NC + c
    rows = (rows0_v, rows1_v)
    sems = (sem0, sem1)

    _zero_rows(rows0_v)
    stripe = s * (_N_PAD // _NS)
    for t in range(5):
        pltpu.sync_copy(rows0_v, acc_sh.at[pl.ds(stripe + t * _W, _W)])
    plsc.subcore_barrier()

    per_w = _E_PAD // _NW
    nbatch = per_w // (_W * _NBW)

    def _batch(b, _):
        base = pl.multiple_of(wid * per_w + b * (_W * _NBW), 128)
        pltpu.sync_copy(src_hbm.at[pl.ds(base, _W * _NBW)], sb_v)
        pltpu.sync_copy(dst_hbm.at[pl.ds(base, _W * _NBW)], db_v)
        pltpu.sync_copy(w_hbm.at[pl.ds(base, _W * _NBW)], wb_v)
        _gather_dma(h_hbm, sb_v, rows[0], sems[0], 0).start()
        _gather_dma(h_hbm, sb_v, rows[1], sems[1], 1).start()
        for slot in range(_NBW):
            buf = slot & 1
            _gather_dma(h_hbm, sb_v, rows[buf], sems[buf], slot).wait()
            for i in range(_W // 16):
                sl = pl.ds(i * 16, 16)
                dst1_v[sl] = db_v[pl.ds(slot * _W + i * 16, 16)]
            _scale_rows(rows[buf], wb_v, slot)
            pltpu.sync_copy(rows[buf], acc_sh.at[dst1_v], add=True)
            if slot + 2 < _NBW:
                _gather_dma(h_hbm, sb_v, rows[buf], sems[buf],
                            slot + 2).start()
        return 0
    lax.fori_loop(0, nbatch, _batch, 0)

    plsc.subcore_barrier()
    pltpu.sync_copy(acc_sh.at[pl.ds(stripe, _N_PAD // _NS)],
                    out_hbm.at[c, pl.ds(stripe, _N_PAD // _NS)])


def _node_segsum(h, src, dst, w):
    mesh = plsc.VectorSubcoreMesh(core_axis_name="c", subcore_axis_name="s")
    return pl.kernel(
        _node_segsum_body,
        out_type=jax.ShapeDtypeStruct((_NC, _N_PAD, _HID), jnp.float32),
        mesh=mesh,
        scratch_types=[
            pltpu.VMEM((_W * _NBW,), jnp.int32),
            pltpu.VMEM((_W * _NBW,), jnp.int32),
            pltpu.VMEM((_W * _NBW,), jnp.float32),
            pltpu.VMEM((_W,), jnp.int32),
            pltpu.VMEM((_W, _HID), jnp.float32),
            pltpu.VMEM((_W, _HID), jnp.float32),
            pltpu.VMEM_SHARED((_N_PAD, _HID), jnp.float32),
            pltpu.SemaphoreType.DMA,
            pltpu.SemaphoreType.DMA,
        ],
    )(h, src, dst, w)


def _node_cochain(x, index, weight, W_emb, W_convs, W_out):
    h = _mm(x, W_emb, act=True)
    h = jnp.concatenate(
        [h, jnp.zeros((_N_PAD - _N_NODES, _HID), jnp.float32)], axis=0)
    npad = _E_PAD - _N_EDGES
    fill = jnp.arange(npad, dtype=jnp.int32)
    src = jnp.concatenate([index[0], fill % _N_NODES])
    dst = jnp.concatenate([index[1], fill % _N_PAD])
    w = jnp.concatenate([weight, jnp.zeros((npad,), jnp.float32)])
    for l in range(W_convs.shape[0] - 1):
        p = _node_segsum(h, src, dst, w)
        h = _mm2(p[0], p[1], W_convs[l])
    return _mm(h, W_out, block=2048)[:_N_NODES]


_L_PAD = 655360
_L_PERM = 818944
_KB = 40
_BROWS = 8192
_PW = _L_PAD // _NW
_STG = 25600
_IB = 1024


def _lap_count_body(dst_hbm, grid_hbm, db_v, hist_v):
    c = lax.axis_index("c")
    s = lax.axis_index("s")
    wid = s * _NC + c
    iota = lax.iota(jnp.int32, 16)
    ones = jnp.ones((16,), jnp.int32)
    for k in range(_KB):
        hist_v[pl.ds(k * 16, 16)] = jnp.zeros((16,), jnp.int32)

    def _batch(bi, _):
        base = pl.multiple_of(wid * _PW + bi * _IB, 128)
        pltpu.sync_copy(dst_hbm.at[pl.ds(base, _IB)], db_v)

        def _c(i, _2):
            d = db_v[pl.ds(i * 16, 16)]
            b = lax.shift_right_logical(d, 13)
            addr = b * 16 + iota
            cur = plsc.load_gather(hist_v, [addr])
            plsc.store_scatter(hist_v, [addr], cur + ones)
            return 0
        lax.fori_loop(0, _IB // 16, _c, 0)
        return 0
    lax.fori_loop(0, _PW // _IB, _batch, 0)
    pltpu.sync_copy(hist_v, grid_hbm.at[wid])


def _lap_count(dst):
    mesh = plsc.VectorSubcoreMesh(core_axis_name="c", subcore_axis_name="s")
    return pl.kernel(
        _lap_count_body,
        compiler_params=pltpu.CompilerParams(needs_layout_passes=False),
        out_type=jax.ShapeDtypeStruct((_NW, _KB * 16), jnp.int32),
        mesh=mesh,
        scratch_types=[
            pltpu.VMEM((_IB,), jnp.int32),
            pltpu.VMEM((_KB * 16,), jnp.int32),
        ],
    )(dst)


def _lap_scan_body(grid_ref, mybase_ref, mycnt_ref, bstart_ref, btotal_ref):
    g = grid_ref[...].astype(jnp.float32)
    js = lax.broadcasted_iota(jnp.int32, (_KB * 16, _KB), 0)
    ks = lax.broadcasted_iota(jnp.int32, (_KB * 16, _KB), 1)
    sel = (lax.shift_right_logical(js, 4) == ks).astype(jnp.float32)
    cnt = jnp.dot(g, sel, preferred_element_type=jnp.float32)
    cnti = cnt.astype(jnp.int32)
    padded = ((cnti + 127) & -128).astype(jnp.float32)
    iw = lax.broadcasted_iota(jnp.int32, (_NW, _NW), 0)
    jw = lax.broadcasted_iota(jnp.int32, (_NW, _NW), 1)
    triw = (iw > jw).astype(jnp.float32)
    wex = jnp.dot(triw, padded, preferred_element_type=jnp.float32)
    ptot = jnp.sum(padded, axis=0, keepdims=True)
    ib = lax.broadcasted_iota(jnp.int32, (_KB, _KB), 0)
    jb = lax.broadcasted_iota(jnp.int32, (_KB, _KB), 1)
    trib = (ib < jb).astype(jnp.float32)
    bex = jnp.dot(ptot, trib, preferred_element_type=jnp.float32)
    mybase = (bex + wex).astype(jnp.int32)
    zpad = jnp.zeros((_NW, 8), jnp.int32)
    mybase_ref[...] = jnp.concatenate([mybase, zpad], axis=1)
    mycnt_ref[...] = jnp.concatenate([cnti, zpad], axis=1)
    dn = (((0,), (0,)), ((), ()))
    ptot_col = lax.dot_general(padded, jnp.ones((_NW, 1), jnp.float32), dn,
                               preferred_element_type=jnp.float32)
    bstart_col = lax.dot_general(trib, ptot_col, dn,
                                 preferred_element_type=jnp.float32)
    bstart_ref[...] = jnp.broadcast_to(bstart_col.astype(jnp.int32),
                                       (_KB, 16))
    btotal_ref[...] = jnp.broadcast_to(ptot_col.astype(jnp.int32), (_KB, 16))


def _lap_scan(grid):
    return pl.pallas_call(
        _lap_scan_body,
        out_shape=(
            jax.ShapeDtypeStruct((_NW, 48), jnp.int32),
            jax.ShapeDtypeStruct((_NW, 48), jnp.int32),
            jax.ShapeDtypeStruct((_KB, 16), jnp.int32),
            jax.ShapeDtypeStruct((_KB, 16), jnp.int32),
        ),
    )(grid)


def _getv(ref, k):
    return ref[pl.ds((k // 16) * 16, 16)][k % 16]


def _lap_perm_body(src_hbm, dst_hbm, w_hbm, grid_hbm, mybase_hbm, mycnt_hbm,
                   srcp_hbm, dstp_hbm, wp_hbm,
                   gridrow_v, mybase_v, mycnt_v, srcb_v, dstb_v, wb_v, ptrs_v,
                   stgs_v, stgd_v, stgw_v):
    c = lax.axis_index("c")
    s = lax.axis_index("s")
    wid = s * _NC + c
    iota = lax.iota(jnp.int32, 16)
    pltpu.sync_copy(grid_hbm.at[wid], gridrow_v)
    roff = pl.multiple_of(wid * 48, 48)
    pltpu.sync_copy(mybase_hbm.at[pl.ds(roff, 48)], mybase_v)
    pltpu.sync_copy(mycnt_hbm.at[pl.ds(roff, 48)], mycnt_v)

    my_base = [_getv(mybase_v, k) for k in range(_KB)]
    my_cnt = [_getv(mycnt_v, k) for k in range(_KB)]
    stg_base = []
    stg_running = jnp.int32(0)
    for k in range(_KB):
        stg_base.append(stg_running)
        lanevec = gridrow_v[pl.ds(k * 16, 16)]
        ex = plsc.cumsum(lanevec) - lanevec + jnp.full((16,), stg_running,
                                                       jnp.int32)
        ptrs_v[pl.ds(k * 16, 16)] = ex
        stg_running = stg_running + ((my_cnt[k] + 127) & -128)

    def _pre(i, _):
        v = jnp.full((16,), i * 16, jnp.int32) + iota
        stgs_v[pl.ds(i * 16, 16)] = v & 0x3FFFF
        stgd_v[pl.ds(i * 16, 16)] = v & 0x1FFF
        stgw_v[pl.ds(i * 16, 16)] = jnp.zeros((16,), jnp.float32)
        return 0
    lax.fori_loop(0, _STG // 16, _pre, 0)

    def _batch(bi, _):
        base = pl.multiple_of(wid * _PW + bi * _IB, 128)
        pltpu.sync_copy(src_hbm.at[pl.ds(base, _IB)], srcb_v)
        pltpu.sync_copy(dst_hbm.at[pl.ds(base, _IB)], dstb_v)
        pltpu.sync_copy(w_hbm.at[pl.ds(base, _IB)], wb_v)

        def _c(i, _2):
            sl = pl.ds(i * 16, 16)
            d = dstb_v[sl]
            b = lax.shift_right_logical(d, 13)
            addr = b * 16 + iota
            pos = plsc.load_gather(ptrs_v, [addr])
            plsc.store_scatter(ptrs_v, [addr], pos + 1)
            plsc.store_scatter(stgs_v, [pos], srcb_v[sl])
            plsc.store_scatter(stgd_v, [pos], d)
            plsc.store_scatter(stgw_v, [pos], wb_v[sl])
            return 0
        lax.fori_loop(0, _IB // 16, _c, 0)
        return 0
    lax.fori_loop(0, _PW // _IB, _batch, 0)

    for k in range(_KB):
        padded = (my_cnt[k] + 127) & -128
        n512 = jnp.right_shift(padded, 9)
        rem = jnp.right_shift(padded, 7) & 3

        def _f5(f, _, k=k):
            so = pl.multiple_of(stg_base[k] + f * 512, 128)
            do = pl.multiple_of(my_base[k] + f * 512, 128)
            pltpu.sync_copy(stgs_v.at[pl.ds(so, 512)],
                            srcp_hbm.at[pl.ds(do, 512)])
            pltpu.sync_copy(stgd_v.at[pl.ds(so, 512)],
                            dstp_hbm.at[pl.ds(do, 512)])
            pltpu.sync_copy(stgw_v.at[pl.ds(so, 512)],
                            wp_hbm.at[pl.ds(do, 512)])
            return 0
        lax.fori_loop(0, n512, _f5, 0)

        def _f1(t, _, k=k, n512=n512):
            tb = n512 * 512 + t * 128
            so = pl.multiple_of(stg_base[k] + tb, 128)
            do = pl.multiple_of(my_base[k] + tb, 128)
            pltpu.sync_copy(stgs_v.at[pl.ds(so, 128)],
                            srcp_hbm.at[pl.ds(do, 128)])
            pltpu.sync_copy(stgd_v.at[pl.ds(so, 128)],
                            dstp_hbm.at[pl.ds(do, 128)])
            pltpu.sync_copy(stgw_v.at[pl.ds(so, 128)],
                            wp_hbm.at[pl.ds(do, 128)])
            return 0
        lax.fori_loop(0, rem, _f1, 0)


def _lap_perm(src, dst, w, grid, mybase, mycnt):
    mesh = plsc.VectorSubcoreMesh(core_axis_name="c", subcore_axis_name="s")
    return pl.kernel(
        _lap_perm_body,
        compiler_params=pltpu.CompilerParams(needs_layout_passes=False),
        out_type=(
            jax.ShapeDtypeStruct((_L_PERM,), jnp.int32),
            jax.ShapeDtypeStruct((_L_PERM,), jnp.int32),
            jax.ShapeDtypeStruct((_L_PERM,), jnp.float32),
        ),
        mesh=mesh,
        scratch_types=[
            pltpu.VMEM((_KB * 16,), jnp.int32),
            pltpu.VMEM((48,), jnp.int32),
            pltpu.VMEM((48,), jnp.int32),
            pltpu.VMEM((_IB,), jnp.int32),
            pltpu.VMEM((_IB,), jnp.int32),
            pltpu.VMEM((_IB,), jnp.float32),
            pltpu.VMEM((_KB * 16,), jnp.int32),
            pltpu.VMEM((_STG,), jnp.int32),
            pltpu.VMEM((_STG,), jnp.int32),
            pltpu.VMEM((_STG,), jnp.float32),
        ],
    )(src, dst, w, grid, mybase, mycnt)


def _lap_segsum_body(h_hbm, srcp_hbm, dstp_hbm, wp_hbm, bsb_hbm, btb_hbm,
                     agg_hbm,
                     bs_v, bt_v, sb_v, db_v, wb_v, ldst_v, rows0_v, rows1_v,
                     acc_sh, sem0, sem1):
    c = lax.axis_index("c")
    s = lax.axis_index("s")
    rows = (rows0_v, rows1_v)
    sems = (sem0, sem1)
    pltpu.sync_copy(bsb_hbm, bs_v)
    pltpu.sync_copy(btb_hbm, bt_v)

    def _bucket(ki, _):
        k = ki * 2 + c
        _zero_rows(rows0_v)

        def _z(t, _2):
            off = pl.multiple_of(s * (_BROWS // _NS) + t * _W, 128)
            pltpu.sync_copy(rows0_v, acc_sh.at[pl.ds(off, _W)])
            return 0
        lax.fori_loop(0, _BROWS // _NS // _W, _z, 0)
        plsc.subcore_barrier()

        bstart = bs_v[pl.ds(k * 16, 16)][0]
        btotal = bt_v[pl.ds(k * 16, 16)][0]
        nw = jnp.right_shift(btotal, 7)
        q = jnp.right_shift(nw, 4)
        r = nw & 15
        lo = s * q + jnp.minimum(s, r)
        cw = q + jnp.where(s < r, 1, 0)
        nb = jnp.right_shift(cw + _NBW - 1, 3)

        def _b8(b, _2):
            gbase = pl.multiple_of(bstart + (lo + b * _NBW) * _W, 128)
            pltpu.sync_copy(srcp_hbm.at[pl.ds(gbase, _W * _NBW)], sb_v)
            pltpu.sync_copy(dstp_hbm.at[pl.ds(gbase, _W * _NBW)], db_v)
            pltpu.sync_copy(wp_hbm.at[pl.ds(gbase, _W * _NBW)], wb_v)

            def _valid(slot):
                return b * _NBW + slot < cw

            def _start(slot, buf):
                _gather_dma(h_hbm, sb_v, rows[buf], sems[buf], slot).start()

            pl.when(_valid(0))(lambda: _start(0, 0))
            pl.when(_valid(1))(lambda: _start(1, 1))
            for slot in range(_NBW):
                buf = slot & 1

                def _do(slot=slot, buf=buf):
                    _gather_dma(h_hbm, sb_v, rows[buf], sems[buf],
                                slot).wait()
                    for i in range(_W // 16):
                        sl = pl.ds(i * 16, 16)
                        ldst_v[sl] = db_v[pl.ds(slot * _W + i * 16,
                                                16)] & 0x1FFF
                    _scale_rows(rows[buf], wb_v, slot)
                    pltpu.sync_copy(rows[buf], acc_sh.at[ldst_v], add=True)
                pl.when(_valid(slot))(_do)
                if slot + 2 < _NBW:
                    pl.when(_valid(slot + 2))(
                        lambda slot=slot, buf=buf: _start(slot + 2, buf))
            return 0
        lax.fori_loop(0, nb, _b8, 0)
        plsc.subcore_barrier()

        rpt = _BROWS // _NS
        pltpu.sync_copy(
            acc_sh.at[pl.ds(pl.multiple_of(s * rpt, rpt), rpt)],
            agg_hbm.at[pl.ds(pl.multiple_of(k * _BROWS + s * rpt, rpt), rpt)])
        return 0
    lax.fori_loop(0, _KB // 2, _bucket, 0)


def _lap_segsum(h, srcp, dstp, wp, bstart_bc, btotal_bc):
    mesh = plsc.VectorSubcoreMesh(core_axis_name="c", subcore_axis_name="s")
    return pl.kernel(
        _lap_segsum_body,
        out_type=jax.ShapeDtypeStruct((_KB * _BROWS, _HID), jnp.float32),
        mesh=mesh,
        scratch_types=[
            pltpu.VMEM((_KB * 16,), jnp.int32),
            pltpu.VMEM((_KB * 16,), jnp.int32),
            pltpu.VMEM((_W * _NBW,), jnp.int32),
            pltpu.VMEM((_W * _NBW,), jnp.int32),
            pltpu.VMEM((_W * _NBW,), jnp.float32),
            pltpu.VMEM((_W,), jnp.int32),
            pltpu.VMEM((_W, _HID), jnp.float32),
            pltpu.VMEM((_W, _HID), jnp.float32),
            pltpu.VMEM_SHARED((_BROWS, _HID), jnp.float32),
            pltpu.SemaphoreType.DMA,
            pltpu.SemaphoreType.DMA,
        ],
    )(h, srcp, dstp, wp, bstart_bc, btotal_bc)


def _edge_cochain(feat, index, weight, W_emb, W_convs, W_out):
    h = _mm(feat, W_emb, act=True, block=8000)
    npad = _L_PAD - 640000
    fill = jnp.arange(npad, dtype=jnp.int32)
    src = jnp.concatenate([index[0], fill % _N_EDGES])
    dst = jnp.concatenate([index[1], fill % _N_EDGES])
    w = jnp.concatenate([weight, jnp.zeros((npad,), jnp.float32)])
    grid = _lap_count(dst)
    mb, mc, bs, bt = _lap_scan(grid)
    srcp, dstp, wp = _lap_perm(src, dst, w, grid,
                               mb.reshape(-1), mc.reshape(-1))
    for l in range(W_convs.shape[0] - 1):
        agg = _lap_segsum(h, srcp, dstp, wp, bs.reshape(-1), bt.reshape(-1))
        h = _mm(agg[:_N_EDGES], W_convs[l], block=8000)
    return _mm(h, W_out, block=8000)


def kernel(x, edge_attr, edge_index, edge_weight, laplacian_index,
           laplacian_weight, W_emb_node, W_convs_node, W_out_node,
           W_emb_edge, W_convs_edge, W_out_edge):
    node_out = _node_cochain(x, edge_index, edge_weight, W_emb_node,
                             W_convs_node, W_out_node)
    edge_out = _edge_cochain(edge_attr, laplacian_index, laplacian_weight,
                             W_emb_edge, W_convs_edge, W_out_edge)
    return (node_out, edge_out)


# speedup vs baseline: 3.1298x; 1.0306x over previous
import functools

import jax
import jax.numpy as jnp
from jax import lax
from jax.experimental import pallas as pl
from jax.experimental.pallas import tpu as pltpu
from jax.experimental.pallas import tpu_sc as plsc

_NC = 2
_NS = 16
_NW = _NC * _NS

_N_NODES = 10000
_N_PAD = 10240
_N_EDGES = 320000
_HID = 128
_W = 128
_NBW = 8
_E_PAD = 327680


def _mm_body(x_ref, w_ref, o_ref, *, act):
    y = jnp.dot(x_ref[...], w_ref[...], preferred_element_type=jnp.float32)
    if act:
        y = jnp.where(y >= 0, y, 0.2 * y)
    o_ref[...] = y


def _mm(x, w, act=False, block=2000):
    n, din = x.shape
    dout = w.shape[1]
    grid = n // block
    return pl.pallas_call(
        functools.partial(_mm_body, act=act),
        grid=(grid,),
        in_specs=[
            pl.BlockSpec((block, din), lambda i: (i, 0)),
            pl.BlockSpec((din, dout), lambda i: (0, 0)),
        ],
        out_specs=pl.BlockSpec((block, dout), lambda i: (i, 0)),
        out_shape=jax.ShapeDtypeStruct((n, dout), jnp.float32),
    )(x, w)


def _mm2_body(a_ref, b_ref, w_ref, o_ref):
    y = jnp.dot(a_ref[...] + b_ref[...], w_ref[...],
                preferred_element_type=jnp.float32)
    o_ref[...] = y


def _mm2(a, b, w, block=2048):
    n, din = a.shape
    dout = w.shape[1]
    grid = n // block
    return pl.pallas_call(
        _mm2_body,
        grid=(grid,),
        in_specs=[
            pl.BlockSpec((block, din), lambda i: (i, 0)),
            pl.BlockSpec((block, din), lambda i: (i, 0)),
            pl.BlockSpec((din, dout), lambda i: (0, 0)),
        ],
        out_specs=pl.BlockSpec((block, dout), lambda i: (i, 0)),
        out_shape=jax.ShapeDtypeStruct((n, dout), jnp.float32),
    )(a, b, w)


def _zero_rows(rows_v):
    def _zb(r, _):
        for j in range(_HID // 16):
            rows_v[r, pl.ds(j * 16, 16)] = jnp.zeros((16,), jnp.float32)
        return 0
    lax.fori_loop(0, _W, _zb, 0)


def _gather_dma(h_hbm, sb_v, rows_v, sem, slot):
    idxr = sb_v.at[pl.ds(slot * _W, _W)]
    return pltpu.make_async_copy(h_hbm.at[idxr], rows_v, sem)


def _scale_rows(rows_v, wb_v, slot):
    def _r16(r16, _):
        wvec = wb_v[pl.ds(slot * _W + r16 * 16, 16)]
        for l in range(16):
            wf = jnp.full((16,), wvec[l], jnp.float32)
            r = r16 * 16 + l
            for j in range(_HID // 16):
                sl = pl.ds(j * 16, 16)
                rows_v[r, sl] = rows_v[r, sl] * wf
        return 0
    lax.fori_loop(0, _W // 16, _r16, 0)


def _node_segsum_body(h_hbm, src_hbm, dst_hbm, w_hbm, out_hbm,
                      sb_v, db_v, wb_v, dst1_v, rows0_v, rows1_v, acc_sh,
                      sem0, sem1):
    c = lax.axis_index("c")
    s = lax.axis_index("s")
    wid = s * _NC + c
    rows = (rows0_v, rows1_v)
    sems = (sem0, sem1)

    _zero_rows(rows0_v)
    stripe = s * (_N_PAD // _NS)
    for t in range(5):
        pltpu.sync_copy(rows0_v, acc_sh.at[pl.ds(stripe + t * _W, _W)])
    plsc.subcore_barrier()

    per_w = _E_PAD // _NW
    nbatch = per_w // (_W * _NBW)

    def _batch(b, _):
        base = pl.multiple_of(wid * per_w + b * (_W * _NBW), 128)
        pltpu.sync_copy(src_hbm.at[pl.ds(base, _W * _NBW)], sb_v)
        pltpu.sync_copy(dst_hbm.at[pl.ds(base, _W * _NBW)], db_v)
        pltpu.sync_copy(w_hbm.at[pl.ds(base, _W * _NBW)], wb_v)
        _gather_dma(h_hbm, sb_v, rows[0], sems[0], 0).start()
        _gather_dma(h_hbm, sb_v, rows[1], sems[1], 1).start()
        for slot in range(_NBW):
            buf = slot & 1
            _gather_dma(h_hbm, sb_v, rows[buf], sems[buf], slot).wait()
            for i in range(_W // 16):
                sl = pl.ds(i * 16, 16)
                dst1_v[sl] = db_v[pl.ds(slot * _W + i * 16, 16)]
            _scale_rows(rows[buf], wb_v, slot)
            pltpu.sync_copy(rows[buf], acc_sh.at[dst1_v], add=True)
            if slot + 2 < _NBW:
                _gather_dma(h_hbm, sb_v, rows[buf], sems[buf],
                            slot + 2).start()
        return 0
    lax.fori_loop(0, nbatch, _batch, 0)

    plsc.subcore_barrier()
    pltpu.sync_copy(acc_sh.at[pl.ds(stripe, _N_PAD // _NS)],
                    out_hbm.at[c, pl.ds(stripe, _N_PAD // _NS)])


def _node_segsum(h, src, dst, w):
    mesh = plsc.VectorSubcoreMesh(core_axis_name="c", subcore_axis_name="s")
    return pl.kernel(
        _node_segsum_body,
        out_type=jax.ShapeDtypeStruct((_NC, _N_PAD, _HID), jnp.float32),
        mesh=mesh,
        scratch_types=[
            pltpu.VMEM((_W * _NBW,), jnp.int32),
            pltpu.VMEM((_W * _NBW,), jnp.int32),
            pltpu.VMEM((_W * _NBW,), jnp.float32),
            pltpu.VMEM((_W,), jnp.int32),
            pltpu.VMEM((_W, _HID), jnp.float32),
            pltpu.VMEM((_W, _HID), jnp.float32),
            pltpu.VMEM_SHARED((_N_PAD, _HID), jnp.float32),
            pltpu.SemaphoreType.DMA,
            pltpu.SemaphoreType.DMA,
        ],
    )(h, src, dst, w)


def _node_cochain(x, index, weight, W_emb, W_convs, W_out):
    h = _mm(x, W_emb, act=True)
    h = jnp.concatenate(
        [h, jnp.zeros((_N_PAD - _N_NODES, _HID), jnp.float32)], axis=0)
    npad = _E_PAD - _N_EDGES
    fill = jnp.arange(npad, dtype=jnp.int32)
    src = jnp.concatenate([index[0], fill % _N_NODES])
    dst = jnp.concatenate([index[1], fill % _N_PAD])
    w = jnp.concatenate([weight, jnp.zeros((npad,), jnp.float32)])
    for l in range(W_convs.shape[0] - 1):
        p = _node_segsum(h, src, dst, w)
        h = _mm2(p[0], p[1], W_convs[l])
    return _mm(h, W_out, block=2048)[:_N_NODES]


_L_PAD = 655360
_L_PERM = 818944
_KB = 40
_BROWS = 8192
_PW = _L_PAD // _NW
_STG = 25600
_IB = 1024


def _lap_count_body(dst_hbm, grid_hbm, db_v, hist_v):
    c = lax.axis_index("c")
    s = lax.axis_index("s")
    wid = s * _NC + c
    iota = lax.iota(jnp.int32, 16)
    ones = jnp.ones((16,), jnp.int32)
    for k in range(_KB):
        hist_v[pl.ds(k * 16, 16)] = jnp.zeros((16,), jnp.int32)

    def _batch(bi, _):
        base = pl.multiple_of(wid * _PW + bi * _IB, 128)
        pltpu.sync_copy(dst_hbm.at[pl.ds(base, _IB)], db_v)

        def _c(i, _2):
            d = db_v[pl.ds(i * 16, 16)]
            b = lax.shift_right_logical(d, 13)
            addr = b * 16 + iota
            cur = plsc.load_gather(hist_v, [addr])
            plsc.store_scatter(hist_v, [addr], cur + ones)
            return 0
        lax.fori_loop(0, _IB // 16, _c, 0)
        return 0
    lax.fori_loop(0, _PW // _IB, _batch, 0)
    pltpu.sync_copy(hist_v, grid_hbm.at[wid])


def _lap_count(dst):
    mesh = plsc.VectorSubcoreMesh(core_axis_name="c", subcore_axis_name="s")
    return pl.kernel(
        _lap_count_body,
        compiler_params=pltpu.CompilerParams(needs_layout_passes=False),
        out_type=jax.ShapeDtypeStruct((_NW, _KB * 16), jnp.int32),
        mesh=mesh,
        scratch_types=[
            pltpu.VMEM((_IB,), jnp.int32),
            pltpu.VMEM((_KB * 16,), jnp.int32),
        ],
    )(dst)


def _lap_scan_body(grid_ref, mybase_ref, mycnt_ref, bstart_ref, btotal_ref):
    g = grid_ref[...].astype(jnp.float32)
    js = lax.broadcasted_iota(jnp.int32, (_KB * 16, _KB), 0)
    ks = lax.broadcasted_iota(jnp.int32, (_KB * 16, _KB), 1)
    sel = (lax.shift_right_logical(js, 4) == ks).astype(jnp.float32)
    cnt = jnp.dot(g, sel, preferred_element_type=jnp.float32)
    cnti = cnt.astype(jnp.int32)
    padded = ((cnti + 127) & -128).astype(jnp.float32)
    iw = lax.broadcasted_iota(jnp.int32, (_NW, _NW), 0)
    jw = lax.broadcasted_iota(jnp.int32, (_NW, _NW), 1)
    triw = (iw > jw).astype(jnp.float32)
    wex = jnp.dot(triw, padded, preferred_element_type=jnp.float32)
    ptot = jnp.sum(padded, axis=0, keepdims=True)
    ib = lax.broadcasted_iota(jnp.int32, (_KB, _KB), 0)
    jb = lax.broadcasted_iota(jnp.int32, (_KB, _KB), 1)
    trib = (ib < jb).astype(jnp.float32)
    bex = jnp.dot(ptot, trib, preferred_element_type=jnp.float32)
    mybase = (bex + wex).astype(jnp.int32)
    zpad = jnp.zeros((_NW, 8), jnp.int32)
    mybase_ref[...] = jnp.concatenate([mybase, zpad], axis=1)
    mycnt_ref[...] = jnp.concatenate([cnti, zpad], axis=1)
    dn = (((0,), (0,)), ((), ()))
    ptot_col = lax.dot_general(padded, jnp.ones((_NW, 1), jnp.float32), dn,
                               preferred_element_type=jnp.float32)
    bstart_col = lax.dot_general(trib, ptot_col, dn,
                                 preferred_element_type=jnp.float32)
    bstart_ref[...] = jnp.broadcast_to(bstart_col.astype(jnp.int32),
                                       (_KB, 16))
    btotal_ref[...] = jnp.broadcast_to(ptot_col.astype(jnp.int32), (_KB, 16))


def _lap_scan(grid):
    return pl.pallas_call(
        _lap_scan_body,
        out_shape=(
            jax.ShapeDtypeStruct((_NW, 48), jnp.int32),
            jax.ShapeDtypeStruct((_NW, 48), jnp.int32),
            jax.ShapeDtypeStruct((_KB, 16), jnp.int32),
            jax.ShapeDtypeStruct((_KB, 16), jnp.int32),
        ),
    )(grid)


def _getv(ref, k):
    return ref[pl.ds((k // 16) * 16, 16)][k % 16]


def _lap_perm_body(src_hbm, dst_hbm, w_hbm, grid_hbm, mybase_hbm, mycnt_hbm,
                   srcp_hbm, dstp_hbm, wp_hbm,
                   gridrow_v, mybase_v, mycnt_v, srcb_v, dstb_v, wb_v, ptrs_v,
                   stgs_v, stgd_v, stgw_v):
    c = lax.axis_index("c")
    s = lax.axis_index("s")
    wid = s * _NC + c
    iota = lax.iota(jnp.int32, 16)
    pltpu.sync_copy(grid_hbm.at[wid], gridrow_v)
    roff = pl.multiple_of(wid * 48, 48)
    pltpu.sync_copy(mybase_hbm.at[pl.ds(roff, 48)], mybase_v)
    pltpu.sync_copy(mycnt_hbm.at[pl.ds(roff, 48)], mycnt_v)

    my_base = [_getv(mybase_v, k) for k in range(_KB)]
    my_cnt = [_getv(mycnt_v, k) for k in range(_KB)]
    stg_base = []
    stg_running = jnp.int32(0)
    for k in range(_KB):
        stg_base.append(stg_running)
        lanevec = gridrow_v[pl.ds(k * 16, 16)]
        ex = plsc.cumsum(lanevec) - lanevec + jnp.full((16,), stg_running,
                                                       jnp.int32)
        ptrs_v[pl.ds(k * 16, 16)] = ex
        stg_running = stg_running + ((my_cnt[k] + 127) & -128)

    def _pre(i, _):
        v = jnp.full((16,), i * 16, jnp.int32) + iota
        stgs_v[pl.ds(i * 16, 16)] = v & 0x3FFFF
        stgd_v[pl.ds(i * 16, 16)] = v & 0x1FFF
        stgw_v[pl.ds(i * 16, 16)] = jnp.zeros((16,), jnp.float32)
        return 0
    lax.fori_loop(0, _STG // 16, _pre, 0)

    def _batch(bi, _):
        base = pl.multiple_of(wid * _PW + bi * _IB, 128)
        pltpu.sync_copy(src_hbm.at[pl.ds(base, _IB)], srcb_v)
        pltpu.sync_copy(dst_hbm.at[pl.ds(base, _IB)], dstb_v)
        pltpu.sync_copy(w_hbm.at[pl.ds(base, _IB)], wb_v)

        def _c(i, _2):
            sl = pl.ds(i * 16, 16)
            d = dstb_v[sl]
            b = lax.shift_right_logical(d, 13)
            addr = b * 16 + iota
            pos = plsc.load_gather(ptrs_v, [addr])
            plsc.store_scatter(ptrs_v, [addr], pos + 1)
            plsc.store_scatter(stgs_v, [pos], srcb_v[sl])
            plsc.store_scatter(stgd_v, [pos], d)
            plsc.store_scatter(stgw_v, [pos], wb_v[sl])
            return 0
        lax.fori_loop(0, _IB // 16, _c, 0)
        return 0
    lax.fori_loop(0, _PW // _IB, _batch, 0)

    for k in range(_KB):
        padded = (my_cnt[k] + 127) & -128
        n512 = jnp.right_shift(padded, 9)
        rem = jnp.right_shift(padded, 7) & 3

        def _f5(f, _, k=k):
            so = pl.multiple_of(stg_base[k] + f * 512, 128)
            do = pl.multiple_of(my_base[k] + f * 512, 128)
            pltpu.sync_copy(stgs_v.at[pl.ds(so, 512)],
                            srcp_hbm.at[pl.ds(do, 512)])
            pltpu.sync_copy(stgd_v.at[pl.ds(so, 512)],
                            dstp_hbm.at[pl.ds(do, 512)])
            pltpu.sync_copy(stgw_v.at[pl.ds(so, 512)],
                            wp_hbm.at[pl.ds(do, 512)])
            return 0
        lax.fori_loop(0, n512, _f5, 0)

        def _f1(t, _, k=k, n512=n512):
            tb = n512 * 512 + t * 128
            so = pl.multiple_of(stg_base[k] + tb, 128)
            do = pl.multiple_of(my_base[k] + tb, 128)
            pltpu.sync_copy(stgs_v.at[pl.ds(so, 128)],
                            srcp_hbm.at[pl.ds(do, 128)])
            pltpu.sync_copy(stgd_v.at[pl.ds(so, 128)],
                            dstp_hbm.at[pl.ds(do, 128)])
            pltpu.sync_copy(stgw_v.at[pl.ds(so, 128)],
                            wp_hbm.at[pl.ds(do, 128)])
            return 0
        lax.fori_loop(0, rem, _f1, 0)


def _lap_perm(src, dst, w, grid, mybase, mycnt):
    mesh = plsc.VectorSubcoreMesh(core_axis_name="c", subcore_axis_name="s")
    return pl.kernel(
        _lap_perm_body,
        compiler_params=pltpu.CompilerParams(needs_layout_passes=False),
        out_type=(
            jax.ShapeDtypeStruct((_L_PERM,), jnp.int32),
            jax.ShapeDtypeStruct((_L_PERM,), jnp.int32),
            jax.ShapeDtypeStruct((_L_PERM,), jnp.float32),
        ),
        mesh=mesh,
        scratch_types=[
            pltpu.VMEM((_KB * 16,), jnp.int32),
            pltpu.VMEM((48,), jnp.int32),
            pltpu.VMEM((48,), jnp.int32),
            pltpu.VMEM((_IB,), jnp.int32),
            pltpu.VMEM((_IB,), jnp.int32),
            pltpu.VMEM((_IB,), jnp.float32),
            pltpu.VMEM((_KB * 16,), jnp.int32),
            pltpu.VMEM((_STG,), jnp.int32),
            pltpu.VMEM((_STG,), jnp.int32),
            pltpu.VMEM((_STG,), jnp.float32),
        ],
    )(src, dst, w, grid, mybase, mycnt)


def _lap_segsum_body(h_hbm, srcp_hbm, dstp_hbm, wp_hbm, bsb_hbm, btb_hbm,
                     agg_hbm,
                     bs_v, bt_v, sb_v, db_v, wb_v,
                     ldst0_v, ldst1_v, ldst2_v, rows0_v, rows1_v, rows2_v,
                     acc_sh, gs0, gs1, gs2, ss0, ss1, ss2):
    c = lax.axis_index("c")
    s = lax.axis_index("s")
    rows = (rows0_v, rows1_v, rows2_v)
    ldst = (ldst0_v, ldst1_v, ldst2_v)
    gsem = (gs0, gs1, gs2)
    ssem = (ss0, ss1, ss2)
    pltpu.sync_copy(bsb_hbm, bs_v)
    pltpu.sync_copy(btb_hbm, bt_v)

    def _scat_dma(buf):
        return pltpu.make_async_copy(rows[buf], acc_sh.at[ldst[buf]],
                                     ssem[buf])

    def _bucket(ki, _):
        k = ki * 2 + c
        _zero_rows(rows0_v)

        def _z(t, _2):
            off = pl.multiple_of(s * (_BROWS // _NS) + t * _W, 128)
            pltpu.sync_copy(rows0_v, acc_sh.at[pl.ds(off, _W)])
            return 0
        lax.fori_loop(0, _BROWS // _NS // _W, _z, 0)
        plsc.subcore_barrier()

        bstart = bs_v[pl.ds(k * 16, 16)][0]
        btotal = bt_v[pl.ds(k * 16, 16)][0]
        nw = jnp.right_shift(btotal, 7)
        q = jnp.right_shift(nw, 4)
        r = nw & 15
        lo = s * q + jnp.minimum(s, r)
        cw = q + jnp.where(s < r, 1, 0)
        nb = jnp.right_shift(cw + _NBW - 1, 3)

        def _b8(b, _2):
            gbase = pl.multiple_of(bstart + (lo + b * _NBW) * _W, 128)
            pltpu.sync_copy(srcp_hbm.at[pl.ds(gbase, _W * _NBW)], sb_v)
            pltpu.sync_copy(dstp_hbm.at[pl.ds(gbase, _W * _NBW)], db_v)
            pltpu.sync_copy(wp_hbm.at[pl.ds(gbase, _W * _NBW)], wb_v)

            def _valid(slot):
                return b * _NBW + slot < cw

            def _gst(slot):
                buf = slot % 3
                _gather_dma(h_hbm, sb_v, rows[buf], gsem[buf], slot).start()

            pl.when(_valid(0))(lambda: _gst(0))
            pl.when(_valid(1))(lambda: _gst(1))
            for slot in range(_NBW):
                buf = slot % 3

                def _do(slot=slot, buf=buf):
                    _gather_dma(h_hbm, sb_v, rows[buf], gsem[buf],
                                slot).wait()
                    for i in range(_W // 16):
                        sl = pl.ds(i * 16, 16)
                        ldst[buf][sl] = db_v[pl.ds(slot * _W + i * 16,
                                                   16)] & 0x1FFF
                    _scale_rows(rows[buf], wb_v, slot)
                    _scat_dma(buf).start(add=True)
                pl.when(_valid(slot))(_do)
                nxt = slot + 2
                if nxt < _NBW:
                    if nxt >= 3:
                        pl.when(_valid(nxt - 3))(
                            lambda nb_=nxt - 3: _scat_dma(nb_ % 3).wait())
                    pl.when(_valid(nxt))(lambda nxt=nxt: _gst(nxt))
            for sl_ in (_NBW - 3, _NBW - 2, _NBW - 1):
                pl.when(_valid(sl_))(
                    lambda sl_=sl_: _scat_dma(sl_ % 3).wait())
            return 0
        lax.fori_loop(0, nb, _b8, 0)
        plsc.subcore_barrier()

        rpt = _BROWS // _NS
        pltpu.sync_copy(
            acc_sh.at[pl.ds(pl.multiple_of(s * rpt, rpt), rpt)],
            agg_hbm.at[pl.ds(pl.multiple_of(k * _BROWS + s * rpt, rpt), rpt)])
        return 0
    lax.fori_loop(0, _KB // 2, _bucket, 0)


def _lap_segsum(h, srcp, dstp, wp, bstart_bc, btotal_bc):
    mesh = plsc.VectorSubcoreMesh(core_axis_name="c", subcore_axis_name="s")
    return pl.kernel(
        _lap_segsum_body,
        out_type=jax.ShapeDtypeStruct((_KB * _BROWS, _HID), jnp.float32),
        mesh=mesh,
        scratch_types=[
            pltpu.VMEM((_KB * 16,), jnp.int32),
            pltpu.VMEM((_KB * 16,), jnp.int32),
            pltpu.VMEM((_W * _NBW,), jnp.int32),
            pltpu.VMEM((_W * _NBW,), jnp.int32),
            pltpu.VMEM((_W * _NBW,), jnp.float32),
            pltpu.VMEM((_W,), jnp.int32),
            pltpu.VMEM((_W,), jnp.int32),
            pltpu.VMEM((_W,), jnp.int32),
            pltpu.VMEM((_W, _HID), jnp.float32),
            pltpu.VMEM((_W, _HID), jnp.float32),
            pltpu.VMEM((_W, _HID), jnp.float32),
            pltpu.VMEM_SHARED((_BROWS, _HID), jnp.float32),
            pltpu.SemaphoreType.DMA,
            pltpu.SemaphoreType.DMA,
            pltpu.SemaphoreType.DMA,
            pltpu.SemaphoreType.DMA,
            pltpu.SemaphoreType.DMA,
            pltpu.SemaphoreType.DMA,
        ],
    )(h, srcp, dstp, wp, bstart_bc, btotal_bc)


def _edge_cochain(feat, index, weight, W_emb, W_convs, W_out):
    h = _mm(feat, W_emb, act=True, block=8000)
    npad = _L_PAD - 640000
    fill = jnp.arange(npad, dtype=jnp.int32)
    src = jnp.concatenate([index[0], fill % _N_EDGES])
    dst = jnp.concatenate([index[1], fill % _N_EDGES])
    w = jnp.concatenate([weight, jnp.zeros((npad,), jnp.float32)])
    grid = _lap_count(dst)
    mb, mc, bs, bt = _lap_scan(grid)
    srcp, dstp, wp = _lap_perm(src, dst, w, grid,
                               mb.reshape(-1), mc.reshape(-1))
    for l in range(W_convs.shape[0] - 1):
        agg = _lap_segsum(h, srcp, dstp, wp, bs.reshape(-1), bt.reshape(-1))
        h = _mm(agg[:_N_EDGES], W_convs[l], block=8000)
    return _mm(h, W_out, block=8000)


def kernel(x, edge_attr, edge_index, edge_weight, laplacian_index,
           laplacian_weight, W_emb_node, W_convs_node, W_out_node,
           W_emb_edge, W_convs_edge, W_out_edge):
    node_out = _node_cochain(x, edge_index, edge_weight, W_emb_node,
                             W_convs_node, W_out_node)
    edge_out = _edge_cochain(edge_attr, laplacian_index, laplacian_weight,
                             W_emb_edge, W_convs_edge, W_out_edge)
    return (node_out, edge_out)


# speedup vs baseline: 3.3199x; 1.0607x over previous
import functools

import jax
import jax.numpy as jnp
from jax import lax
from jax.experimental import pallas as pl
from jax.experimental.pallas import tpu as pltpu
from jax.experimental.pallas import tpu_sc as plsc

_NC = 2
_NS = 16
_NW = _NC * _NS

_N_NODES = 10000
_N_PAD = 10240
_N_EDGES = 320000
_HID = 128
_W = 128
_NBW = 16
_E_PAD = 327680


def _mm_body(x_ref, w_ref, o_ref, *, act):
    y = jnp.dot(x_ref[...], w_ref[...], preferred_element_type=jnp.float32)
    if act:
        y = jnp.where(y >= 0, y, 0.2 * y)
    o_ref[...] = y


def _mm(x, w, act=False, block=2000):
    n, din = x.shape
    dout = w.shape[1]
    grid = n // block
    return pl.pallas_call(
        functools.partial(_mm_body, act=act),
        grid=(grid,),
        in_specs=[
            pl.BlockSpec((block, din), lambda i: (i, 0)),
            pl.BlockSpec((din, dout), lambda i: (0, 0)),
        ],
        out_specs=pl.BlockSpec((block, dout), lambda i: (i, 0)),
        out_shape=jax.ShapeDtypeStruct((n, dout), jnp.float32),
    )(x, w)


def _mm2_body(a_ref, b_ref, w_ref, o_ref):
    y = jnp.dot(a_ref[...] + b_ref[...], w_ref[...],
                preferred_element_type=jnp.float32)
    o_ref[...] = y


def _mm2(a, b, w, block=2048):
    n, din = a.shape
    dout = w.shape[1]
    grid = n // block
    return pl.pallas_call(
        _mm2_body,
        grid=(grid,),
        in_specs=[
            pl.BlockSpec((block, din), lambda i: (i, 0)),
            pl.BlockSpec((block, din), lambda i: (i, 0)),
            pl.BlockSpec((din, dout), lambda i: (0, 0)),
        ],
        out_specs=pl.BlockSpec((block, dout), lambda i: (i, 0)),
        out_shape=jax.ShapeDtypeStruct((n, dout), jnp.float32),
    )(a, b, w)


def _zero_rows(rows_v):
    def _zb(r, _):
        for j in range(_HID // 16):
            rows_v[r, pl.ds(j * 16, 16)] = jnp.zeros((16,), jnp.float32)
        return 0
    lax.fori_loop(0, _W, _zb, 0)


def _gather_dma(h_hbm, sb_v, rows_v, sem, slot):
    idxr = sb_v.at[pl.ds(slot * _W, _W)]
    return pltpu.make_async_copy(h_hbm.at[idxr], rows_v, sem)


def _scale_rows(rows_v, wb_v, slot):
    def _r16(r16, _):
        wvec = wb_v[pl.ds(slot * _W + r16 * 16, 16)]
        for l in range(16):
            wf = jnp.full((16,), wvec[l], jnp.float32)
            r = r16 * 16 + l
            for j in range(_HID // 16):
                sl = pl.ds(j * 16, 16)
                rows_v[r, sl] = rows_v[r, sl] * wf
        return 0
    lax.fori_loop(0, _W // 16, _r16, 0)


def _node_segsum_body(h_hbm, src_hbm, dst_hbm, w_hbm, out_hbm,
                      sb_v, db_v, wb_v, dst1_v, rows0_v, rows1_v, acc_sh,
                      sem0, sem1):
    c = lax.axis_index("c")
    s = lax.axis_index("s")
    wid = s * _NC + c
    rows = (rows0_v, rows1_v)
    sems = (sem0, sem1)

    _zero_rows(rows0_v)
    stripe = s * (_N_PAD // _NS)
    for t in range(5):
        pltpu.sync_copy(rows0_v, acc_sh.at[pl.ds(stripe + t * _W, _W)])
    plsc.subcore_barrier()

    per_w = _E_PAD // _NW
    nbatch = per_w // (_W * _NBW)

    def _batch(b, _):
        base = pl.multiple_of(wid * per_w + b * (_W * _NBW), 128)
        pltpu.sync_copy(src_hbm.at[pl.ds(base, _W * _NBW)], sb_v)
        pltpu.sync_copy(dst_hbm.at[pl.ds(base, _W * _NBW)], db_v)
        pltpu.sync_copy(w_hbm.at[pl.ds(base, _W * _NBW)], wb_v)
        _gather_dma(h_hbm, sb_v, rows[0], sems[0], 0).start()
        _gather_dma(h_hbm, sb_v, rows[1], sems[1], 1).start()
        for slot in range(_NBW):
            buf = slot & 1
            _gather_dma(h_hbm, sb_v, rows[buf], sems[buf], slot).wait()
            for i in range(_W // 16):
                sl = pl.ds(i * 16, 16)
                dst1_v[sl] = db_v[pl.ds(slot * _W + i * 16, 16)]
            _scale_rows(rows[buf], wb_v, slot)
            pltpu.sync_copy(rows[buf], acc_sh.at[dst1_v], add=True)
            if slot + 2 < _NBW:
                _gather_dma(h_hbm, sb_v, rows[buf], sems[buf],
                            slot + 2).start()
        return 0
    lax.fori_loop(0, nbatch, _batch, 0)

    plsc.subcore_barrier()
    pltpu.sync_copy(acc_sh.at[pl.ds(stripe, _N_PAD // _NS)],
                    out_hbm.at[c, pl.ds(stripe, _N_PAD // _NS)])


def _node_segsum(h, src, dst, w):
    mesh = plsc.VectorSubcoreMesh(core_axis_name="c", subcore_axis_name="s")
    return pl.kernel(
        _node_segsum_body,
        out_type=jax.ShapeDtypeStruct((_NC, _N_PAD, _HID), jnp.float32),
        mesh=mesh,
        scratch_types=[
            pltpu.VMEM((_W * _NBW,), jnp.int32),
            pltpu.VMEM((_W * _NBW,), jnp.int32),
            pltpu.VMEM((_W * _NBW,), jnp.float32),
            pltpu.VMEM((_W,), jnp.int32),
            pltpu.VMEM((_W, _HID), jnp.float32),
            pltpu.VMEM((_W, _HID), jnp.float32),
            pltpu.VMEM_SHARED((_N_PAD, _HID), jnp.float32),
            pltpu.SemaphoreType.DMA,
            pltpu.SemaphoreType.DMA,
        ],
    )(h, src, dst, w)


def _node_cochain(x, index, weight, W_emb, W_convs, W_out):
    h = _mm(x, W_emb, act=True)
    h = jnp.concatenate(
        [h, jnp.zeros((_N_PAD - _N_NODES, _HID), jnp.float32)], axis=0)
    npad = _E_PAD - _N_EDGES
    fill = jnp.arange(npad, dtype=jnp.int32)
    src = jnp.concatenate([index[0], fill % _N_NODES])
    dst = jnp.concatenate([index[1], fill % _N_PAD])
    w = jnp.concatenate([weight, jnp.zeros((npad,), jnp.float32)])
    for l in range(W_convs.shape[0] - 1):
        p = _node_segsum(h, src, dst, w)
        h = _mm2(p[0], p[1], W_convs[l])
    return _mm(h, W_out, block=2048)[:_N_NODES]


_L_PAD = 655360
_L_PERM = 818944
_KB = 40
_BROWS = 8192
_PW = _L_PAD // _NW
_STG = 25600
_IB = 1024


def _lap_count_body(dst_hbm, grid_hbm, db_v, hist_v):
    c = lax.axis_index("c")
    s = lax.axis_index("s")
    wid = s * _NC + c
    iota = lax.iota(jnp.int32, 16)
    ones = jnp.ones((16,), jnp.int32)
    for k in range(_KB):
        hist_v[pl.ds(k * 16, 16)] = jnp.zeros((16,), jnp.int32)

    def _batch(bi, _):
        base = pl.multiple_of(wid * _PW + bi * _IB, 128)
        pltpu.sync_copy(dst_hbm.at[pl.ds(base, _IB)], db_v)

        def _c(i, _2):
            d = db_v[pl.ds(i * 16, 16)]
            b = lax.shift_right_logical(d, 13)
            addr = b * 16 + iota
            cur = plsc.load_gather(hist_v, [addr])
            plsc.store_scatter(hist_v, [addr], cur + ones)
            return 0
        lax.fori_loop(0, _IB // 16, _c, 0)
        return 0
    lax.fori_loop(0, _PW // _IB, _batch, 0)
    pltpu.sync_copy(hist_v, grid_hbm.at[wid])


def _lap_count(dst):
    mesh = plsc.VectorSubcoreMesh(core_axis_name="c", subcore_axis_name="s")
    return pl.kernel(
        _lap_count_body,
        compiler_params=pltpu.CompilerParams(needs_layout_passes=False),
        out_type=jax.ShapeDtypeStruct((_NW, _KB * 16), jnp.int32),
        mesh=mesh,
        scratch_types=[
            pltpu.VMEM((_IB,), jnp.int32),
            pltpu.VMEM((_KB * 16,), jnp.int32),
        ],
    )(dst)


def _lap_scan_body(grid_ref, mybase_ref, mycnt_ref, bstart_ref, btotal_ref):
    g = grid_ref[...].astype(jnp.float32)
    js = lax.broadcasted_iota(jnp.int32, (_KB * 16, _KB), 0)
    ks = lax.broadcasted_iota(jnp.int32, (_KB * 16, _KB), 1)
    sel = (lax.shift_right_logical(js, 4) == ks).astype(jnp.float32)
    cnt = jnp.dot(g, sel, preferred_element_type=jnp.float32)
    cnti = cnt.astype(jnp.int32)
    padded = ((cnti + 127) & -128).astype(jnp.float32)
    iw = lax.broadcasted_iota(jnp.int32, (_NW, _NW), 0)
    jw = lax.broadcasted_iota(jnp.int32, (_NW, _NW), 1)
    triw = (iw > jw).astype(jnp.float32)
    wex = jnp.dot(triw, padded, preferred_element_type=jnp.float32)
    ptot = jnp.sum(padded, axis=0, keepdims=True)
    ib = lax.broadcasted_iota(jnp.int32, (_KB, _KB), 0)
    jb = lax.broadcasted_iota(jnp.int32, (_KB, _KB), 1)
    trib = (ib < jb).astype(jnp.float32)
    bex = jnp.dot(ptot, trib, preferred_element_type=jnp.float32)
    mybase = (bex + wex).astype(jnp.int32)
    zpad = jnp.zeros((_NW, 8), jnp.int32)
    mybase_ref[...] = jnp.concatenate([mybase, zpad], axis=1)
    mycnt_ref[...] = jnp.concatenate([cnti, zpad], axis=1)
    dn = (((0,), (0,)), ((), ()))
    ptot_col = lax.dot_general(padded, jnp.ones((_NW, 1), jnp.float32), dn,
                               preferred_element_type=jnp.float32)
    bstart_col = lax.dot_general(trib, ptot_col, dn,
                                 preferred_element_type=jnp.float32)
    bstart_ref[...] = jnp.broadcast_to(bstart_col.astype(jnp.int32),
                                       (_KB, 16))
    btotal_ref[...] = jnp.broadcast_to(ptot_col.astype(jnp.int32), (_KB, 16))


def _lap_scan(grid):
    return pl.pallas_call(
        _lap_scan_body,
        out_shape=(
            jax.ShapeDtypeStruct((_NW, 48), jnp.int32),
            jax.ShapeDtypeStruct((_NW, 48), jnp.int32),
            jax.ShapeDtypeStruct((_KB, 16), jnp.int32),
            jax.ShapeDtypeStruct((_KB, 16), jnp.int32),
        ),
    )(grid)


def _getv(ref, k):
    return ref[pl.ds((k // 16) * 16, 16)][k % 16]


def _lap_perm_body(src_hbm, dst_hbm, w_hbm, grid_hbm, mybase_hbm, mycnt_hbm,
                   srcp_hbm, dstp_hbm, wp_hbm,
                   gridrow_v, mybase_v, mycnt_v, srcb_v, dstb_v, wb_v, ptrs_v,
                   stgs_v, stgd_v, stgw_v):
    c = lax.axis_index("c")
    s = lax.axis_index("s")
    wid = s * _NC + c
    iota = lax.iota(jnp.int32, 16)
    pltpu.sync_copy(grid_hbm.at[wid], gridrow_v)
    roff = pl.multiple_of(wid * 48, 48)
    pltpu.sync_copy(mybase_hbm.at[pl.ds(roff, 48)], mybase_v)
    pltpu.sync_copy(mycnt_hbm.at[pl.ds(roff, 48)], mycnt_v)

    my_base = [_getv(mybase_v, k) for k in range(_KB)]
    my_cnt = [_getv(mycnt_v, k) for k in range(_KB)]
    stg_base = []
    stg_running = jnp.int32(0)
    for k in range(_KB):
        stg_base.append(stg_running)
        lanevec = gridrow_v[pl.ds(k * 16, 16)]
        ex = plsc.cumsum(lanevec) - lanevec + jnp.full((16,), stg_running,
                                                       jnp.int32)
        ptrs_v[pl.ds(k * 16, 16)] = ex
        stg_running = stg_running + ((my_cnt[k] + 127) & -128)

    def _pre(i, _):
        v = jnp.full((16,), i * 16, jnp.int32) + iota
        stgs_v[pl.ds(i * 16, 16)] = v & 0x3FFFF
        stgd_v[pl.ds(i * 16, 16)] = v & 0x1FFF
        stgw_v[pl.ds(i * 16, 16)] = jnp.zeros((16,), jnp.float32)
        return 0
    lax.fori_loop(0, _STG // 16, _pre, 0)

    def _batch(bi, _):
        base = pl.multiple_of(wid * _PW + bi * _IB, 128)
        pltpu.sync_copy(src_hbm.at[pl.ds(base, _IB)], srcb_v)
        pltpu.sync_copy(dst_hbm.at[pl.ds(base, _IB)], dstb_v)
        pltpu.sync_copy(w_hbm.at[pl.ds(base, _IB)], wb_v)

        def _c(i, _2):
            sl = pl.ds(i * 16, 16)
            d = dstb_v[sl]
            b = lax.shift_right_logical(d, 13)
            addr = b * 16 + iota
            pos = plsc.load_gather(ptrs_v, [addr])
            plsc.store_scatter(ptrs_v, [addr], pos + 1)
            plsc.store_scatter(stgs_v, [pos], srcb_v[sl])
            plsc.store_scatter(stgd_v, [pos], d)
            plsc.store_scatter(stgw_v, [pos], wb_v[sl])
            return 0
        lax.fori_loop(0, _IB // 16, _c, 0)
        return 0
    lax.fori_loop(0, _PW // _IB, _batch, 0)

    for k in range(_KB):
        padded = (my_cnt[k] + 127) & -128
        n512 = jnp.right_shift(padded, 9)
        rem = jnp.right_shift(padded, 7) & 3

        def _f5(f, _, k=k):
            so = pl.multiple_of(stg_base[k] + f * 512, 128)
            do = pl.multiple_of(my_base[k] + f * 512, 128)
            pltpu.sync_copy(stgs_v.at[pl.ds(so, 512)],
                            srcp_hbm.at[pl.ds(do, 512)])
            pltpu.sync_copy(stgd_v.at[pl.ds(so, 512)],
                            dstp_hbm.at[pl.ds(do, 512)])
            pltpu.sync_copy(stgw_v.at[pl.ds(so, 512)],
                            wp_hbm.at[pl.ds(do, 512)])
            return 0
        lax.fori_loop(0, n512, _f5, 0)

        def _f1(t, _, k=k, n512=n512):
            tb = n512 * 512 + t * 128
            so = pl.multiple_of(stg_base[k] + tb, 128)
            do = pl.multiple_of(my_base[k] + tb, 128)
            pltpu.sync_copy(stgs_v.at[pl.ds(so, 128)],
                            srcp_hbm.at[pl.ds(do, 128)])
            pltpu.sync_copy(stgd_v.at[pl.ds(so, 128)],
                            dstp_hbm.at[pl.ds(do, 128)])
            pltpu.sync_copy(stgw_v.at[pl.ds(so, 128)],
                            wp_hbm.at[pl.ds(do, 128)])
            return 0
        lax.fori_loop(0, rem, _f1, 0)


def _lap_perm(src, dst, w, grid, mybase, mycnt):
    mesh = plsc.VectorSubcoreMesh(core_axis_name="c", subcore_axis_name="s")
    return pl.kernel(
        _lap_perm_body,
        compiler_params=pltpu.CompilerParams(needs_layout_passes=False),
        out_type=(
            jax.ShapeDtypeStruct((_L_PERM,), jnp.int32),
            jax.ShapeDtypeStruct((_L_PERM,), jnp.int32),
            jax.ShapeDtypeStruct((_L_PERM,), jnp.float32),
        ),
        mesh=mesh,
        scratch_types=[
            pltpu.VMEM((_KB * 16,), jnp.int32),
            pltpu.VMEM((48,), jnp.int32),
            pltpu.VMEM((48,), jnp.int32),
            pltpu.VMEM((_IB,), jnp.int32),
            pltpu.VMEM((_IB,), jnp.int32),
            pltpu.VMEM((_IB,), jnp.float32),
            pltpu.VMEM((_KB * 16,), jnp.int32),
            pltpu.VMEM((_STG,), jnp.int32),
            pltpu.VMEM((_STG,), jnp.int32),
            pltpu.VMEM((_STG,), jnp.float32),
        ],
    )(src, dst, w, grid, mybase, mycnt)


def _lap_segsum_body(h_hbm, srcp_hbm, dstp_hbm, wp_hbm, bsb_hbm, btb_hbm,
                     agg_hbm,
                     bs_v, bt_v, sb_v, db_v, wb_v,
                     ldst0_v, ldst1_v, ldst2_v, rows0_v, rows1_v, rows2_v,
                     acc_sh, gs0, gs1, gs2, ss0, ss1, ss2):
    c = lax.axis_index("c")
    s = lax.axis_index("s")
    rows = (rows0_v, rows1_v, rows2_v)
    ldst = (ldst0_v, ldst1_v, ldst2_v)
    gsem = (gs0, gs1, gs2)
    ssem = (ss0, ss1, ss2)
    pltpu.sync_copy(bsb_hbm, bs_v)
    pltpu.sync_copy(btb_hbm, bt_v)

    def _scat_dma(buf):
        return pltpu.make_async_copy(rows[buf], acc_sh.at[ldst[buf]],
                                     ssem[buf])

    def _bucket(ki, _):
        k = ki * 2 + c
        _zero_rows(rows0_v)

        def _z(t, _2):
            off = pl.multiple_of(s * (_BROWS // _NS) + t * _W, 128)
            pltpu.sync_copy(rows0_v, acc_sh.at[pl.ds(off, _W)])
            return 0
        lax.fori_loop(0, _BROWS // _NS // _W, _z, 0)
        plsc.subcore_barrier()

        bstart = bs_v[pl.ds(k * 16, 16)][0]
        btotal = bt_v[pl.ds(k * 16, 16)][0]
        nw = jnp.right_shift(btotal, 7)
        q = jnp.right_shift(nw, 4)
        r = nw & 15
        lo = s * q + jnp.minimum(s, r)
        cw = q + jnp.where(s < r, 1, 0)
        nb = jnp.right_shift(cw + _NBW - 1, 4)

        def _b8(b, _2):
            gbase = pl.multiple_of(bstart + (lo + b * _NBW) * _W, 128)
            pltpu.sync_copy(srcp_hbm.at[pl.ds(gbase, _W * _NBW)], sb_v)
            pltpu.sync_copy(dstp_hbm.at[pl.ds(gbase, _W * _NBW)], db_v)
            pltpu.sync_copy(wp_hbm.at[pl.ds(gbase, _W * _NBW)], wb_v)

            def _valid(slot):
                return b * _NBW + slot < cw

            def _gst(slot):
                buf = slot % 3
                _gather_dma(h_hbm, sb_v, rows[buf], gsem[buf], slot).start()

            pl.when(_valid(0))(lambda: _gst(0))
            pl.when(_valid(1))(lambda: _gst(1))
            for slot in range(_NBW):
                buf = slot % 3

                def _do(slot=slot, buf=buf):
                    _gather_dma(h_hbm, sb_v, rows[buf], gsem[buf],
                                slot).wait()
                    for i in range(_W // 16):
                        sl = pl.ds(i * 16, 16)
                        ldst[buf][sl] = db_v[pl.ds(slot * _W + i * 16,
                                                   16)] & 0x1FFF
                    _scale_rows(rows[buf], wb_v, slot)
                    _scat_dma(buf).start(add=True)
                pl.when(_valid(slot))(_do)
                nxt = slot + 2
                if nxt < _NBW:
                    if nxt >= 3:
                        pl.when(_valid(nxt - 3))(
                            lambda nb_=nxt - 3: _scat_dma(nb_ % 3).wait())
                    pl.when(_valid(nxt))(lambda nxt=nxt: _gst(nxt))
            for sl_ in (_NBW - 3, _NBW - 2, _NBW - 1):
                pl.when(_valid(sl_))(
                    lambda sl_=sl_: _scat_dma(sl_ % 3).wait())
            return 0
        lax.fori_loop(0, nb, _b8, 0)
        plsc.subcore_barrier()

        rpt = _BROWS // _NS
        pltpu.sync_copy(
            acc_sh.at[pl.ds(pl.multiple_of(s * rpt, rpt), rpt)],
            agg_hbm.at[pl.ds(pl.multiple_of(k * _BROWS + s * rpt, rpt), rpt)])
        return 0
    lax.fori_loop(0, _KB // 2, _bucket, 0)


def _lap_segsum(h, srcp, dstp, wp, bstart_bc, btotal_bc):
    mesh = plsc.VectorSubcoreMesh(core_axis_name="c", subcore_axis_name="s")
    return pl.kernel(
        _lap_segsum_body,
        out_type=jax.ShapeDtypeStruct((_KB * _BROWS, _HID), jnp.float32),
        mesh=mesh,
        scratch_types=[
            pltpu.VMEM((_KB * 16,), jnp.int32),
            pltpu.VMEM((_KB * 16,), jnp.int32),
            pltpu.VMEM((_W * _NBW,), jnp.int32),
            pltpu.VMEM((_W * _NBW,), jnp.int32),
            pltpu.VMEM((_W * _NBW,), jnp.float32),
            pltpu.VMEM((_W,), jnp.int32),
            pltpu.VMEM((_W,), jnp.int32),
            pltpu.VMEM((_W,), jnp.int32),
            pltpu.VMEM((_W, _HID), jnp.float32),
            pltpu.VMEM((_W, _HID), jnp.float32),
            pltpu.VMEM((_W, _HID), jnp.float32),
            pltpu.VMEM_SHARED((_BROWS, _HID), jnp.float32),
            pltpu.SemaphoreType.DMA,
            pltpu.SemaphoreType.DMA,
            pltpu.SemaphoreType.DMA,
            pltpu.SemaphoreType.DMA,
            pltpu.SemaphoreType.DMA,
            pltpu.SemaphoreType.DMA,
        ],
    )(h, srcp, dstp, wp, bstart_bc, btotal_bc)


def _edge_cochain(feat, index, weight, W_emb, W_convs, W_out):
    h = _mm(feat, W_emb, act=True, block=8000)
    npad = _L_PAD - 640000
    fill = jnp.arange(npad, dtype=jnp.int32)
    src = jnp.concatenate([index[0], fill % _N_EDGES])
    dst = jnp.concatenate([index[1], fill % _N_EDGES])
    w = jnp.concatenate([weight, jnp.zeros((npad,), jnp.float32)])
    grid = _lap_count(dst)
    mb, mc, bs, bt = _lap_scan(grid)
    srcp, dstp, wp = _lap_perm(src, dst, w, grid,
                               mb.reshape(-1), mc.reshape(-1))
    for l in range(W_convs.shape[0] - 1):
        agg = _lap_segsum(h, srcp, dstp, wp, bs.reshape(-1), bt.reshape(-1))
        h = _mm(agg[:_N_EDGES], W_convs[l], block=8000)
    return _mm(h, W_out, block=8000)


def kernel(x, edge_attr, edge_index, edge_weight, laplacian_index,
           laplacian_weight, W_emb_node, W_convs_node, W_out_node,
           W_emb_edge, W_convs_edge, W_out_edge):
    node_out = _node_cochain(x, edge_index, edge_weight, W_emb_node,
                             W_convs_node, W_out_node)
    edge_out = _edge_cochain(edge_attr, laplacian_index, laplacian_weight,
                             W_emb_edge, W_convs_edge, W_out_edge)
    return (node_out, edge_out)
